# baseline (speedup 1.0000x reference)
import functools

import jax
import jax.numpy as jnp
from jax import lax
from jax.experimental import pallas as pl
from jax.experimental.pallas import tpu as pltpu
from jax.experimental.pallas import tpu_sc as plsc

N = 10000
D = 128
E = 320000

NC = 2
NS = 16
L = 16
NP = 10240
SB = 160
BB = SB // NC
W = 4
EP = NS * SB * 128
RPT = NP // NS
NB = EP // 128


def _newton_rsqrt(s):
    s = jnp.maximum(s, 1e-24)
    i = lax.bitcast_convert_type(s, jnp.int32)
    i = jnp.int32(0x5F3759DF) - lax.shift_right_logical(i, 1)
    y = lax.bitcast_convert_type(i, jnp.float32)
    for _ in range(4):
        y = y * (1.5 - 0.5 * s * y * y)
    return y


def _xn_body(x_ref, o_ref):
    xb = x_ref[...]
    s = jnp.sum(xb * xb, axis=1, keepdims=True)
    o_ref[...] = xb * lax.rsqrt(jnp.maximum(s, 1e-24))


def _combine_body(ws_ref, wi_ref, xn_ref, p0_ref, p1_ref, o_ref):
    o_ref[...] = (ws_ref[...] * xn_ref[...]
                  + wi_ref[...] * (p0_ref[...] + p1_ref[...]))


def _sc_body(row_hbm, col_hbm, ea_hbm, beta_hbm, eps_hbm, xn_hbm,
             part_hbm, ws_hbm, wi_hbm,
             row_win, ea_win, col_win, rsq_v, slice_v, wbuf,
             beta_v, eps_v, gbuf,
             sq_s, asum_s, acc_s, gsem, wsem, ssem):
    c = lax.axis_index("c")
    s = lax.axis_index("s")
    base = s * RPT
    abase = s * SB
    gbase = s * SB + c * BB

    pltpu.sync_copy(beta_hbm, beta_v)
    pltpu.sync_copy(eps_hbm, eps_v)
    bvec = beta_v[...]
    ebeta = jnp.exp(bvec)

    for k in range(RPT // L):
        slice_v[pl.ds(k * L, L)] = jnp.zeros((L,), jnp.float32)
    pltpu.sync_copy(slice_v, sq_s.at[pl.ds(base, RPT)])

    def _zrow(j, carry):
        for k in range(D // L):
            gbuf[0, j, pl.ds(k * L, L)] = jnp.zeros((L,), jnp.float32)
        return carry
    lax.fori_loop(0, 128, _zrow, 0)
    for t in range(RPT // 128):
        pltpu.sync_copy(gbuf.at[0], acc_s.at[pl.ds(base + t * 128, 128)])

    for k in range(RPT // L):
        slice_v[pl.ds(k * L, L)] = ebeta
    pltpu.sync_copy(slice_v, asum_s.at[pl.ds(base, RPT)])
    plsc.subcore_barrier()

    pltpu.sync_copy(row_hbm.at[pl.ds(abase, W)], row_win.at[0])
    pltpu.sync_copy(ea_hbm.at[pl.ds(abase, W)], ea_win.at[0])

    def _a1(t, carry):
        p = t % 2

        @pl.when(t + 1 < SB // W)
        def _():
            nxt = pl.ds(abase + (t + 1) * W, W)
            pltpu.async_copy(row_hbm.at[nxt], row_win.at[1 - p], wsem)
            pltpu.async_copy(ea_hbm.at[nxt], ea_win.at[1 - p], wsem)
        for b in range(W):
            for g in range(8):
                eav = ea_win[p, b, pl.ds(g * L, L)]
                wbuf[b, pl.ds(g * L, L)] = eav * eav
        for b in range(W):
            pltpu.async_copy(wbuf.at[b], sq_s.at[row_win.at[p, b]], ssem,
                             add=True)
        for b in range(W):
            pltpu.make_async_copy(wbuf.at[b], sq_s.at[row_win.at[p, b]],
                                  ssem).wait()

        @pl.when(t + 1 < SB // W)
        def _():
            nxt = pl.ds(abase + (t + 1) * W, W)
            pltpu.make_async_copy(row_hbm.at[nxt], row_win.at[1 - p],
                                  wsem).wait()
            pltpu.make_async_copy(ea_hbm.at[nxt], ea_win.at[1 - p],
                                  wsem).wait()
        return carry
    lax.fori_loop(0, SB // W, _a1, 0)
    plsc.subcore_barrier()

    pltpu.sync_copy(sq_s.at[pl.ds(base, RPT)], slice_v)
    for k in range(RPT // L):
        slice_v[pl.ds(k * L, L)] = _newton_rsqrt(slice_v[pl.ds(k * L, L)])
    pltpu.sync_copy(slice_v, sq_s.at[pl.ds(base, RPT)])
    plsc.subcore_barrier()
    pltpu.sync_copy(sq_s, rsq_v)

    pltpu.sync_copy(row_hbm.at[pl.ds(abase, W)], row_win.at[0])
    pltpu.sync_copy(ea_hbm.at[pl.ds(abase, W)], ea_win.at[0])

    def _a2(t, carry):
        p = t % 2

        @pl.when(t + 1 < SB // W)
        def _():
            nxt = pl.ds(abase + (t + 1) * W, W)
            pltpu.async_copy(row_hbm.at[nxt], row_win.at[1 - p], wsem)
            pltpu.async_copy(ea_hbm.at[nxt], ea_win.at[1 - p], wsem)
        for b in range(W):
            for g in range(8):
                rowv = row_win[p, b, pl.ds(g * L, L)]
                eav = ea_win[p, b, pl.ds(g * L, L)]
                rsqv = plsc.load_gather(rsq_v, [rowv])
                wbuf[b, pl.ds(g * L, L)] = jnp.exp(bvec * eav * rsqv)
        for b in range(W):
            pltpu.async_copy(wbuf.at[b], asum_s.at[row_win.at[p, b]], ssem,
                             add=True)
        for b in range(W):
            pltpu.make_async_copy(wbuf.at[b], asum_s.at[row_win.at[p, b]],
                                  ssem).wait()

        @pl.when(t + 1 < SB // W)
        def _():
            nxt = pl.ds(abase + (t + 1) * W, W)
            pltpu.make_async_copy(row_hbm.at[nxt], row_win.at[1 - p],
                                  wsem).wait()
            pltpu.make_async_copy(ea_hbm.at[nxt], ea_win.at[1 - p],
                                  wsem).wait()
        return carry
    lax.fori_loop(0, SB // W, _a2, 0)
    plsc.subcore_barrier()

    pltpu.sync_copy(asum_s.at[pl.ds(base, RPT)], slice_v)
    epsv = eps_v[...]
    for k in range(RPT // L):
        slice_v[pl.ds(k * L, L)] = 1.0 / (slice_v[pl.ds(k * L, L)] + 1e-16)

    @pl.when(c == 0)
    def _():
        pltpu.sync_copy(slice_v, wi_hbm.at[pl.ds(base, RPT)])
    for k in range(RPT // L):
        slice_v[pl.ds(k * L, L)] = (1.0 + epsv) + ebeta * slice_v[pl.ds(k * L, L)]

    @pl.when(c == 0)
    def _():
        pltpu.sync_copy(slice_v, ws_hbm.at[pl.ds(base, RPT)])

    pltpu.sync_copy(row_hbm.at[pl.ds(gbase, W)], row_win.at[0])
    pltpu.sync_copy(ea_hbm.at[pl.ds(gbase, W)], ea_win.at[0])
    pltpu.sync_copy(col_hbm.at[pl.ds(gbase, W)], col_win.at[0])
    pltpu.async_copy(xn_hbm.at[col_win.at[0, 0]], gbuf.at[0], gsem)

    def _bwin(t, carry):
        p = t % 2
        @pl.when(t + 1 < BB // W)
        def _():
            nxt = pl.ds(gbase + (t + 1) * W, W)
            pltpu.sync_copy(row_hbm.at[nxt], row_win.at[1 - p])
            pltpu.sync_copy(ea_hbm.at[nxt], ea_win.at[1 - p])
            pltpu.sync_copy(col_hbm.at[nxt], col_win.at[1 - p])
        for b in range(W):
            jb = t * W + b
            q = jb % 2
            @pl.when(jb + 1 < BB)
            def _():
                pn = p if b + 1 < W else 1 - p
                bn = (b + 1) % W
                pltpu.async_copy(xn_hbm.at[col_win.at[pn, bn]],
                                 gbuf.at[1 - q], gsem)
            for g in range(8):
                rowv = row_win[p, b, pl.ds(g * L, L)]
                eav = ea_win[p, b, pl.ds(g * L, L)]
                rsqv = plsc.load_gather(rsq_v, [rowv])
                wbuf[0, pl.ds(g * L, L)] = jnp.exp(bvec * eav * rsqv)
            pltpu.make_async_copy(xn_hbm.at[col_win.at[p, b]],
                                  gbuf.at[q], gsem).wait()

            def _grpfn(g, carry2):
                wv = wbuf[0, pl.ds(g * L, L)]
                for u in range(L):
                    wb = jnp.broadcast_to(wv[u], (L,))
                    j = g * L + u
                    for k in range(D // L):
                        gbuf[q, j, pl.ds(k * L, L)] = \
                            gbuf[q, j, pl.ds(k * L, L)] * wb
                return carry2
            lax.fori_loop(0, 8, _grpfn, 0)
            pltpu.sync_copy(gbuf.at[q], acc_s.at[row_win.at[p, b]],
                            add=True)
        return carry
    lax.fori_loop(0, BB // W, _bwin, 0)
    plsc.subcore_barrier()

    pltpu.sync_copy(acc_s.at[pl.ds(base, RPT)],
                    part_hbm.at[c, pl.ds(base, RPT)])


_sc_kernel = functools.partial(
    pl.kernel,
    out_type=(
        jax.ShapeDtypeStruct((NC, NP, D), jnp.float32),
        jax.ShapeDtypeStruct((NP,), jnp.float32),
        jax.ShapeDtypeStruct((NP,), jnp.float32),
    ),
    mesh=plsc.VectorSubcoreMesh(core_axis_name="c", subcore_axis_name="s"),
    compiler_params=pltpu.CompilerParams(needs_layout_passes=False),
    scratch_types=[
        pltpu.VMEM((2, W, 128), jnp.int32),
        pltpu.VMEM((2, W, 128), jnp.float32),
        pltpu.VMEM((2, W, 128), jnp.int32),
        pltpu.VMEM((NP,), jnp.float32),
        pltpu.VMEM((RPT,), jnp.float32),
        pltpu.VMEM((W, 128), jnp.float32),
        pltpu.VMEM((L,), jnp.float32),
        pltpu.VMEM((L,), jnp.float32),
        pltpu.VMEM((2, 128, D), jnp.float32),
        pltpu.VMEM_SHARED((NP,), jnp.float32),
        pltpu.VMEM_SHARED((NP,), jnp.float32),
        pltpu.VMEM_SHARED((NP, D), jnp.float32),
        pltpu.SemaphoreType.DMA,
        pltpu.SemaphoreType.DMA,
        pltpu.SemaphoreType.DMA,
    ],
)(_sc_body)


@jax.jit
def kernel(x, edge_index, edge_attr, beta, eps):
    row = edge_index[0]
    col = edge_index[1]
    pad = EP - E
    pad_row = N + (jnp.arange(pad, dtype=jnp.int32) % (NP - N))
    pad_col = jnp.arange(pad, dtype=jnp.int32) % N
    row_p = jnp.concatenate([row, pad_row]).reshape(NB, 128)
    col_p = jnp.concatenate([col, pad_col]).reshape(NB, 128)
    ea_p = jnp.concatenate(
        [edge_attr, jnp.zeros((pad,), jnp.float32)]).reshape(NB, 128)
    beta16 = jnp.broadcast_to(beta.astype(jnp.float32), (L,))
    eps16 = jnp.broadcast_to(eps.astype(jnp.float32), (L,))

    xn = pl.pallas_call(
        _xn_body,
        grid=(N // 200,),
        in_specs=[pl.BlockSpec((200, D), lambda i: (i, 0))],
        out_specs=pl.BlockSpec((200, D), lambda i: (i, 0)),
        out_shape=jax.ShapeDtypeStruct((N, D), jnp.float32),
    )(x)

    part, wscale, winv = _sc_kernel(row_p, col_p, ea_p, beta16, eps16, xn)

    out = pl.pallas_call(
        _combine_body,
        grid=(N // 200,),
        in_specs=[
            pl.BlockSpec((200, 1), lambda i: (i, 0)),
            pl.BlockSpec((200, 1), lambda i: (i, 0)),
            pl.BlockSpec((200, D), lambda i: (i, 0)),
            pl.BlockSpec((200, D), lambda i: (i, 0)),
            pl.BlockSpec((200, D), lambda i: (i, 0)),
        ],
        out_specs=pl.BlockSpec((200, D), lambda i: (i, 0)),
        out_shape=jax.ShapeDtypeStruct((N, D), jnp.float32),
    )(wscale.reshape(NP, 1), winv.reshape(NP, 1), xn, part[0], part[1])

    return out


# speedup vs baseline: 1.1596x; 1.1596x over previous
import functools

import jax
import jax.numpy as jnp
from jax import lax
from jax.experimental import pallas as pl
from jax.experimental.pallas import tpu as pltpu
from jax.experimental.pallas import tpu_sc as plsc

N = 10000
D = 128
E = 320000

NC = 2
NS = 16
L = 16
NP = 10240
SB = 160
BB = SB // NC
W = 4
EP = NS * SB * 128
RPT = NP // NS
NB = EP // 128


def _newton_rsqrt(s):
    s = jnp.maximum(s, 1e-24)
    i = lax.bitcast_convert_type(s, jnp.int32)
    i = jnp.int32(0x5F3759DF) - lax.shift_right_logical(i, 1)
    y = lax.bitcast_convert_type(i, jnp.float32)
    for _ in range(4):
        y = y * (1.5 - 0.5 * s * y * y)
    return y


def _xn_body(x_ref, o_ref):
    xb = x_ref[...]
    s = jnp.sum(xb * xb, axis=1, keepdims=True)
    o_ref[...] = xb * lax.rsqrt(jnp.maximum(s, 1e-24))


def _combine_body(ws_ref, wi_ref, xn_ref, p0_ref, p1_ref, o_ref):
    o_ref[...] = (ws_ref[...] * xn_ref[...]
                  + wi_ref[...] * (p0_ref[...] + p1_ref[...]))


def _sc_body(row_hbm, col_hbm, ea_hbm, beta_hbm, eps_hbm, xn_hbm,
             p0_hbm, p1_hbm, ws_hbm, wi_hbm,
             row_win, ea_win, col_win, rsq_v, slice_v, wbuf,
             beta_v, eps_v, gbuf,
             sq_s, asum_s, acc_s, gsem, wsem, ssem):
    c = lax.axis_index("c")
    s = lax.axis_index("s")
    base = s * RPT
    abase = s * SB
    gbase = s * SB + c * BB

    pltpu.sync_copy(beta_hbm, beta_v)
    pltpu.sync_copy(eps_hbm, eps_v)
    bvec = beta_v[...]
    ebeta = jnp.exp(bvec)

    for k in range(RPT // L):
        slice_v[pl.ds(k * L, L)] = jnp.zeros((L,), jnp.float32)
    pltpu.sync_copy(slice_v, sq_s.at[pl.ds(base, RPT)])

    def _zrow(j, carry):
        for k in range(D // L):
            gbuf[0, j, pl.ds(k * L, L)] = jnp.zeros((L,), jnp.float32)
        return carry
    lax.fori_loop(0, 128, _zrow, 0)
    for t in range(RPT // 128):
        pltpu.sync_copy(gbuf.at[0], acc_s.at[pl.ds(base + t * 128, 128)])

    for k in range(RPT // L):
        slice_v[pl.ds(k * L, L)] = ebeta
    pltpu.sync_copy(slice_v, asum_s.at[pl.ds(base, RPT)])
    plsc.subcore_barrier()

    pltpu.sync_copy(row_hbm.at[pl.ds(abase, W)], row_win.at[0])
    pltpu.sync_copy(ea_hbm.at[pl.ds(abase, W)], ea_win.at[0])

    def _a1(t, carry):
        p = t % 2

        @pl.when(t + 1 < SB // W)
        def _():
            nxt = pl.ds(abase + (t + 1) * W, W)
            pltpu.async_copy(row_hbm.at[nxt], row_win.at[1 - p], wsem)
            pltpu.async_copy(ea_hbm.at[nxt], ea_win.at[1 - p], wsem)
        for b in range(W):
            for g in range(8):
                eav = ea_win[p, b, pl.ds(g * L, L)]
                wbuf[b, pl.ds(g * L, L)] = eav * eav
        for b in range(W):
            pltpu.async_copy(wbuf.at[b], sq_s.at[row_win.at[p, b]], ssem,
                             add=True)
        for b in range(W):
            pltpu.make_async_copy(wbuf.at[b], sq_s.at[row_win.at[p, b]],
                                  ssem).wait()

        @pl.when(t + 1 < SB // W)
        def _():
            nxt = pl.ds(abase + (t + 1) * W, W)
            pltpu.make_async_copy(row_hbm.at[nxt], row_win.at[1 - p],
                                  wsem).wait()
            pltpu.make_async_copy(ea_hbm.at[nxt], ea_win.at[1 - p],
                                  wsem).wait()
        return carry
    lax.fori_loop(0, SB // W, _a1, 0)
    plsc.subcore_barrier()

    pltpu.sync_copy(sq_s.at[pl.ds(base, RPT)], slice_v)
    for k in range(RPT // L):
        slice_v[pl.ds(k * L, L)] = _newton_rsqrt(slice_v[pl.ds(k * L, L)])
    pltpu.sync_copy(slice_v, sq_s.at[pl.ds(base, RPT)])
    plsc.subcore_barrier()
    pltpu.sync_copy(sq_s, rsq_v)

    pltpu.sync_copy(row_hbm.at[pl.ds(abase, W)], row_win.at[0])
    pltpu.sync_copy(ea_hbm.at[pl.ds(abase, W)], ea_win.at[0])

    def _a2(t, carry):
        p = t % 2

        @pl.when(t + 1 < SB // W)
        def _():
            nxt = pl.ds(abase + (t + 1) * W, W)
            pltpu.async_copy(row_hbm.at[nxt], row_win.at[1 - p], wsem)
            pltpu.async_copy(ea_hbm.at[nxt], ea_win.at[1 - p], wsem)
        for b in range(W):
            for g in range(8):
                rowv = row_win[p, b, pl.ds(g * L, L)]
                eav = ea_win[p, b, pl.ds(g * L, L)]
                rsqv = plsc.load_gather(rsq_v, [rowv])
                wbuf[b, pl.ds(g * L, L)] = jnp.exp(bvec * eav * rsqv)
        for b in range(W):
            pltpu.async_copy(wbuf.at[b], asum_s.at[row_win.at[p, b]], ssem,
                             add=True)
        for b in range(W):
            pltpu.make_async_copy(wbuf.at[b], asum_s.at[row_win.at[p, b]],
                                  ssem).wait()

        @pl.when(t + 1 < SB // W)
        def _():
            nxt = pl.ds(abase + (t + 1) * W, W)
            pltpu.make_async_copy(row_hbm.at[nxt], row_win.at[1 - p],
                                  wsem).wait()
            pltpu.make_async_copy(ea_hbm.at[nxt], ea_win.at[1 - p],
                                  wsem).wait()
        return carry
    lax.fori_loop(0, SB // W, _a2, 0)
    plsc.subcore_barrier()

    pltpu.sync_copy(asum_s.at[pl.ds(base, RPT)], slice_v)
    epsv = eps_v[...]
    for k in range(RPT // L):
        slice_v[pl.ds(k * L, L)] = 1.0 / (slice_v[pl.ds(k * L, L)] + 1e-16)

    @pl.when(c == 0)
    def _():
        pltpu.sync_copy(slice_v, wi_hbm.at[pl.ds(base, RPT)])
    for k in range(RPT // L):
        slice_v[pl.ds(k * L, L)] = (1.0 + epsv) + ebeta * slice_v[pl.ds(k * L, L)]

    @pl.when(c == 0)
    def _():
        pltpu.sync_copy(slice_v, ws_hbm.at[pl.ds(base, RPT)])

    pltpu.sync_copy(row_hbm.at[pl.ds(gbase, W)], row_win.at[0])
    pltpu.sync_copy(ea_hbm.at[pl.ds(gbase, W)], ea_win.at[0])
    pltpu.sync_copy(col_hbm.at[pl.ds(gbase, W)], col_win.at[0])
    pltpu.async_copy(xn_hbm.at[col_win.at[0, 0]], gbuf.at[0], gsem)

    def _bwin(t, carry):
        p = t % 2
        @pl.when(t + 1 < BB // W)
        def _():
            nxt = pl.ds(gbase + (t + 1) * W, W)
            pltpu.sync_copy(row_hbm.at[nxt], row_win.at[1 - p])
            pltpu.sync_copy(ea_hbm.at[nxt], ea_win.at[1 - p])
            pltpu.sync_copy(col_hbm.at[nxt], col_win.at[1 - p])
        for b in range(W):
            jb = t * W + b
            q = jb % 2
            @pl.when(jb + 1 < BB)
            def _():
                pn = p if b + 1 < W else 1 - p
                bn = (b + 1) % W
                pltpu.async_copy(xn_hbm.at[col_win.at[pn, bn]],
                                 gbuf.at[1 - q], gsem)
            for g in range(8):
                rowv = row_win[p, b, pl.ds(g * L, L)]
                eav = ea_win[p, b, pl.ds(g * L, L)]
                rsqv = plsc.load_gather(rsq_v, [rowv])
                wbuf[0, pl.ds(g * L, L)] = jnp.exp(bvec * eav * rsqv)
            pltpu.make_async_copy(xn_hbm.at[col_win.at[p, b]],
                                  gbuf.at[q], gsem).wait()

            def _grpfn(g, carry2):
                wv = wbuf[0, pl.ds(g * L, L)]
                for u in range(L):
                    wb = jnp.broadcast_to(wv[u], (L,))
                    j = g * L + u
                    for k in range(D // L):
                        gbuf[q, j, pl.ds(k * L, L)] = \
                            gbuf[q, j, pl.ds(k * L, L)] * wb
                return carry2
            lax.fori_loop(0, 8, _grpfn, 0)
            pltpu.sync_copy(gbuf.at[q], acc_s.at[row_win.at[p, b]],
                            add=True)
        return carry
    lax.fori_loop(0, BB // W, _bwin, 0)
    plsc.subcore_barrier()

    @pl.when(c == 0)
    def _():
        pltpu.sync_copy(acc_s.at[pl.ds(base, RPT)],
                        p0_hbm.at[pl.ds(base, RPT)])

    @pl.when(c == 1)
    def _():
        pltpu.sync_copy(acc_s.at[pl.ds(base, RPT)],
                        p1_hbm.at[pl.ds(base, RPT)])


_sc_kernel = functools.partial(
    pl.kernel,
    out_type=(
        jax.ShapeDtypeStruct((NP, D), jnp.float32),
        jax.ShapeDtypeStruct((NP, D), jnp.float32),
        jax.ShapeDtypeStruct((NP,), jnp.float32),
        jax.ShapeDtypeStruct((NP,), jnp.float32),
    ),
    mesh=plsc.VectorSubcoreMesh(core_axis_name="c", subcore_axis_name="s"),
    compiler_params=pltpu.CompilerParams(needs_layout_passes=False),
    scratch_types=[
        pltpu.VMEM((2, W, 128), jnp.int32),
        pltpu.VMEM((2, W, 128), jnp.float32),
        pltpu.VMEM((2, W, 128), jnp.int32),
        pltpu.VMEM((NP,), jnp.float32),
        pltpu.VMEM((RPT,), jnp.float32),
        pltpu.VMEM((W, 128), jnp.float32),
        pltpu.VMEM((L,), jnp.float32),
        pltpu.VMEM((L,), jnp.float32),
        pltpu.VMEM((2, 128, D), jnp.float32),
        pltpu.VMEM_SHARED((NP,), jnp.float32),
        pltpu.VMEM_SHARED((NP,), jnp.float32),
        pltpu.VMEM_SHARED((NP, D), jnp.float32),
        pltpu.SemaphoreType.DMA,
        pltpu.SemaphoreType.DMA,
        pltpu.SemaphoreType.DMA,
    ],
)(_sc_body)


@jax.jit
def kernel(x, edge_index, edge_attr, beta, eps):
    row = edge_index[0]
    col = edge_index[1]
    pad = EP - E
    pad_row = N + (jnp.arange(pad, dtype=jnp.int32) % (NP - N))
    pad_col = jnp.arange(pad, dtype=jnp.int32) % N
    row_p = jnp.concatenate([row, pad_row]).reshape(NB, 128)
    col_p = jnp.concatenate([col, pad_col]).reshape(NB, 128)
    ea_p = jnp.concatenate(
        [edge_attr, jnp.zeros((pad,), jnp.float32)]).reshape(NB, 128)
    beta16 = jnp.broadcast_to(beta.astype(jnp.float32), (L,))
    eps16 = jnp.broadcast_to(eps.astype(jnp.float32), (L,))

    xn = pl.pallas_call(
        _xn_body,
        grid=(N // 1000,),
        in_specs=[pl.BlockSpec((1000, D), lambda i: (i, 0))],
        out_specs=pl.BlockSpec((1000, D), lambda i: (i, 0)),
        out_shape=jax.ShapeDtypeStruct((N, D), jnp.float32),
    )(x)

    p0, p1, wscale, winv = _sc_kernel(row_p, col_p, ea_p, beta16, eps16, xn)

    out = pl.pallas_call(
        _combine_body,
        grid=(N // 1000,),
        in_specs=[
            pl.BlockSpec((1000, 1), lambda i: (i, 0)),
            pl.BlockSpec((1000, 1), lambda i: (i, 0)),
            pl.BlockSpec((1000, D), lambda i: (i, 0)),
            pl.BlockSpec((1000, D), lambda i: (i, 0)),
            pl.BlockSpec((1000, D), lambda i: (i, 0)),
        ],
        out_specs=pl.BlockSpec((1000, D), lambda i: (i, 0)),
        out_shape=jax.ShapeDtypeStruct((N, D), jnp.float32),
    )(wscale.reshape(NP, 1), winv.reshape(NP, 1), xn, p0, p1)

    return out
